# baseline (speedup 1.0000x reference)
import jax
import jax.numpy as jnp
from jax.experimental import pallas as pl
from jax.experimental.pallas import tpu as pltpu

_LONGEST_LENGTH = 4096


def _pad_body(x_ref, o_ref, zbuf, copy_sem, zero_sem):
    B, T, D = x_ref.shape
    zbuf[...] = jnp.zeros_like(zbuf)
    copies = [
        pltpu.make_async_copy(x_ref.at[b], o_ref.at[b, 0:T, :], copy_sem)
        for b in range(B)
    ]
    zeros = [
        pltpu.make_async_copy(zbuf, o_ref.at[b, T:, :], zero_sem)
        for b in range(B)
    ]
    for c in copies:
        c.start()
    for z in zeros:
        z.start()
    for c in copies:
        c.wait()
    for z in zeros:
        z.wait()


def kernel(x, input_sizes_list=None, longest_length=None):
    B, T, D = x.shape
    L = _LONGEST_LENGTH
    out = pl.pallas_call(
        _pad_body,
        in_specs=[pl.BlockSpec(memory_space=pl.ANY)],
        out_specs=pl.BlockSpec(memory_space=pl.ANY),
        out_shape=jax.ShapeDtypeStruct((B, L, D), x.dtype),
        scratch_shapes=[
            pltpu.VMEM((L - T, D), x.dtype),
            pltpu.SemaphoreType.DMA,
            pltpu.SemaphoreType.DMA,
        ],
    )(x)
    return out


# speedup vs baseline: 13.1133x; 13.1133x over previous
import jax
import jax.numpy as jnp
from jax.experimental import pallas as pl

_LONGEST_LENGTH = 4096
_BT = 256


def _pad_body(x_ref, o_ref, *, n_copy):
    t = pl.program_id(1)

    @pl.when(t < n_copy)
    def _copy():
        o_ref[...] = x_ref[...]

    @pl.when(t >= n_copy)
    def _zero():
        o_ref[...] = jnp.zeros_like(o_ref)


def kernel(x, input_sizes_list=None, longest_length=None):
    B, T, D = x.shape
    L = _LONGEST_LENGTH
    bt = _BT
    n_copy = T // bt
    import functools
    body = functools.partial(_pad_body, n_copy=n_copy)
    out = pl.pallas_call(
        body,
        grid=(B, L // bt),
        in_specs=[
            pl.BlockSpec((1, bt, D), lambda b, t: (b, jnp.minimum(t, T // _BT - 1), 0))
        ],
        out_specs=pl.BlockSpec((1, bt, D), lambda b, t: (b, t, 0)),
        out_shape=jax.ShapeDtypeStruct((B, L, D), x.dtype),
    )(x)
    return out


# speedup vs baseline: 32.1775x; 2.4538x over previous
import jax
import jax.numpy as jnp
from jax.experimental import pallas as pl

_LONGEST_LENGTH = 4096


def _pad_body(x_ref, o_ref):
    T = x_ref.shape[1]
    o_ref[:, :T, :] = x_ref[...]
    o_ref[:, T:, :] = jnp.zeros_like(o_ref[:, T:, :])


def kernel(x, input_sizes_list=None, longest_length=None):
    B, T, D = x.shape
    L = _LONGEST_LENGTH
    out = pl.pallas_call(
        _pad_body,
        grid=(B,),
        in_specs=[pl.BlockSpec((1, T, D), lambda b: (b, 0, 0))],
        out_specs=pl.BlockSpec((1, L, D), lambda b: (b, 0, 0)),
        out_shape=jax.ShapeDtypeStruct((B, L, D), x.dtype),
    )(x)
    return out


# speedup vs baseline: 33.1658x; 1.0307x over previous
import jax
import jax.numpy as jnp
from jax.experimental import pallas as pl

_LONGEST_LENGTH = 4096


def _pad_body(x_ref, o_ref):
    T = x_ref.shape[1]
    o_ref[:, :T, :] = x_ref[...]
    o_ref[:, T:, :] = jnp.zeros_like(o_ref[:, T:, :])


def kernel(x, input_sizes_list=None, longest_length=None):
    B, T, D = x.shape
    L = _LONGEST_LENGTH
    bb = 2
    out = pl.pallas_call(
        _pad_body,
        grid=(B // bb,),
        in_specs=[pl.BlockSpec((bb, T, D), lambda b: (b, 0, 0))],
        out_specs=pl.BlockSpec((bb, L, D), lambda b: (b, 0, 0)),
        out_shape=jax.ShapeDtypeStruct((B, L, D), x.dtype),
    )(x)
    return out


# speedup vs baseline: 33.1852x; 1.0006x over previous
import jax
import jax.numpy as jnp
from jax.experimental import pallas as pl
from jax.experimental.pallas import tpu as pltpu

_LONGEST_LENGTH = 4096


def _pad_body(x_ref, o_ref):
    T = x_ref.shape[1]
    o_ref[:, :T, :] = x_ref[...]
    o_ref[:, T:, :] = jnp.zeros_like(o_ref[:, T:, :])


def kernel(x, input_sizes_list=None, longest_length=None):
    B, T, D = x.shape
    L = _LONGEST_LENGTH
    bb = 2
    out = pl.pallas_call(
        _pad_body,
        grid=(B // bb,),
        in_specs=[pl.BlockSpec((bb, T, D), lambda b: (b, 0, 0))],
        out_specs=pl.BlockSpec((bb, L, D), lambda b: (b, 0, 0)),
        out_shape=jax.ShapeDtypeStruct((B, L, D), x.dtype),
        compiler_params=pltpu.CompilerParams(
            vmem_limit_bytes=120 * 1024 * 1024,
        ),
    )(x)
    return out


# speedup vs baseline: 33.2898x; 1.0032x over previous
import jax
import jax.numpy as jnp
from jax.experimental import pallas as pl
from jax.experimental.pallas import tpu as pltpu

_LONGEST_LENGTH = 4096


def _pad_body(x_ref, o_ref):
    T = x_ref.shape[1]
    o_ref[:, :T, :] = x_ref[...]
    o_ref[:, T:, :] = jnp.zeros_like(o_ref[:, T:, :])


def kernel(x, input_sizes_list=None, longest_length=None):
    B, T, D = x.shape
    L = _LONGEST_LENGTH
    bb = 2 if B % 2 == 0 else 1
    out = pl.pallas_call(
        _pad_body,
        grid=(B // bb,),
        in_specs=[pl.BlockSpec((bb, T, D), lambda b: (b, 0, 0))],
        out_specs=pl.BlockSpec((bb, L, D), lambda b: (b, 0, 0)),
        out_shape=jax.ShapeDtypeStruct((B, L, D), x.dtype),
        compiler_params=pltpu.CompilerParams(
            vmem_limit_bytes=120 * 1024 * 1024,
        ),
    )(x)
    return out
